# baseline (speedup 1.0000x reference)
import functools

import jax
import jax.numpy as jnp
from jax import lax
from jax.experimental import pallas as pl
from jax.experimental.pallas import tpu as pltpu
from jax.experimental.pallas import tpu_sc as plsc

_L = 16
_CH = 16
_NW = 32


def _rsqrt16(v):
    i = lax.bitcast_convert_type(v, jnp.int32)
    i = jnp.int32(0x5F3759DF) - lax.shift_right_arithmetic(i, 1)
    y = lax.bitcast_convert_type(i, jnp.float32)
    half = v * 0.5
    for _ in range(3):
        y = y * (1.5 - half * y * y)
    return y


def _xlane_sum(a):
    idx = lax.iota(jnp.int32, _L)
    dnums = lax.GatherDimensionNumbers(
        offset_dims=(), collapsed_slice_dims=(0,), start_index_map=(0,))
    for sh in (8, 4, 2, 1):
        perm = lax.bitwise_xor(idx, sh)
        a = a + lax.gather(a, perm[:, None], dnums, (1,),
                           mode=lax.GatherScatterMode.PROMISE_IN_BOUNDS)
    return a


def _sc_ln_body(S, x_hbm, w_hbm, g_hbm, b_hbm, out_hbm, wbuf, gbuf, bbuf, xbuf):
    N, D = x_hbm.shape
    nj = D // _L
    rpw = N // _NW
    nch = rpw // _CH

    wid = lax.axis_index("c") * 16 + lax.axis_index("s")
    row0 = wid * rpw
    bidx = row0 // S

    pltpu.sync_copy(w_hbm.at[pl.ds(bidx, 1)], wbuf)
    pltpu.sync_copy(g_hbm, gbuf)
    pltpu.sync_copy(b_hbm, bbuf)

    def chunk_body(c, _):
        r0 = row0 + c * _CH
        pltpu.sync_copy(x_hbm.at[pl.ds(r0, _CH)], xbuf)

        def row_body(i, _):
            acc_s = jnp.zeros((_L,), jnp.float32)
            acc_q = jnp.zeros((_L,), jnp.float32)
            for j in range(nj):
                v = xbuf[i, pl.ds(j * _L, _L)] + wbuf[0, pl.ds(j * _L, _L)]
                acc_s = acc_s + v
                acc_q = acc_q + v * v
            mean_v = _xlane_sum(acc_s) * (1.0 / D)
            q_v = _xlane_sum(acc_q) * (1.0 / D)
            var_v = jnp.maximum(q_v - mean_v * mean_v, 0.0)
            inv_v = _rsqrt16(var_v + 1e-9)
            for j in range(nj):
                v = xbuf[i, pl.ds(j * _L, _L)] + wbuf[0, pl.ds(j * _L, _L)]
                g = gbuf[pl.ds(j * _L, _L)]
                bta = bbuf[pl.ds(j * _L, _L)]
                xbuf[i, pl.ds(j * _L, _L)] = (v - mean_v) * inv_v * g + bta
            return 0

        lax.fori_loop(0, _CH, row_body, 0)
        pltpu.sync_copy(xbuf, out_hbm.at[pl.ds(r0, _CH)])
        return 0

    lax.fori_loop(0, nch, chunk_body, 0)


def kernel(x, W, gamma, beta):
    B, S, D = x.shape
    N = B * S
    x2 = x.reshape(N, D)
    mesh = plsc.VectorSubcoreMesh(core_axis_name="c", subcore_axis_name="s")
    out = pl.kernel(
        functools.partial(_sc_ln_body, S),
        mesh=mesh,
        out_type=jax.ShapeDtypeStruct((N, D), jnp.float32),
        scratch_types=[
            pltpu.VMEM((1, D), jnp.float32),
            pltpu.VMEM((D,), jnp.float32),
            pltpu.VMEM((D,), jnp.float32),
            pltpu.VMEM((_CH, D), jnp.float32),
        ],
    )(x2, W, gamma, beta)
    return out.reshape(B, S, D)


# speedup vs baseline: 9.9825x; 9.9825x over previous
import jax
import jax.numpy as jnp
from jax.experimental import pallas as pl
from jax.experimental.pallas import tpu as pltpu

_BLK = 2048


def _ln_kernel(x_ref, w_ref, g_ref, b_ref, o_ref):
    x = x_ref[0]
    e = w_ref[0, 0]
    y = x + e[None, :]
    mean = jnp.mean(y, axis=1, keepdims=True)
    yc = y - mean
    var = jnp.mean(yc * yc, axis=1, keepdims=True)
    inv = jax.lax.rsqrt(var + 1e-9)
    o_ref[0] = yc * inv * g_ref[0][None, :] + b_ref[0][None, :]


def kernel(x, W, gamma, beta):
    B, S, D = x.shape
    W3 = W[:B].reshape(B, 1, D)
    g2 = gamma.reshape(1, D)
    b2 = beta.reshape(1, D)
    grid = (B, S // _BLK)
    return pl.pallas_call(
        _ln_kernel,
        grid=grid,
        in_specs=[
            pl.BlockSpec((1, _BLK, D), lambda b, s: (b, s, 0)),
            pl.BlockSpec((1, 1, D), lambda b, s: (b, 0, 0)),
            pl.BlockSpec((1, D), lambda b, s: (0, 0)),
            pl.BlockSpec((1, D), lambda b, s: (0, 0)),
        ],
        out_specs=pl.BlockSpec((1, _BLK, D), lambda b, s: (b, s, 0)),
        out_shape=jax.ShapeDtypeStruct((B, S, D), x.dtype),
        compiler_params=pltpu.CompilerParams(
            dimension_semantics=("parallel", "parallel"),
        ),
    )(x, W3, g2, b2)
